# baseline (speedup 1.0000x reference)
import functools

import jax
import jax.numpy as jnp
from jax import lax
from jax.experimental import pallas as pl
from jax.experimental.pallas import tpu as pltpu
from jax.experimental.pallas import tpu_sc as plsc

_LANES = 16


def _build(n_rows: int, n_cols: int, n_workers: int):
    rows_per = n_rows // n_workers
    per = rows_per * n_cols
    n_chunks = rows_per // _LANES
    assert rows_per * n_workers == n_rows and n_chunks * _LANES == rows_per
    mesh = plsc.VectorSubcoreMesh(core_axis_name="c", subcore_axis_name="s")

    @functools.partial(
        pl.kernel,
        out_type=jax.ShapeDtypeStruct((n_rows * n_cols,), jnp.float32),
        mesh=mesh,
        scratch_types=[
            pltpu.VMEM((_LANES,), jnp.float32),
            pltpu.VMEM((n_cols, rows_per), jnp.int32),
            pltpu.VMEM((per,), jnp.float32),
            pltpu.SemaphoreType.DMA,
            pltpu.SemaphoreType.DMA,
            pltpu.SemaphoreType.DMA,
        ],
        compiler_params=pltpu.CompilerParams(
            needs_layout_passes=False, skip_device_barrier=True),
    )
    def run(xt_hbm, w_hbm, out_hbm, tab, xts, ov, s1, s2, s3):
        wid = lax.axis_index("s") * 2 + lax.axis_index("c")
        row0 = wid * rows_per
        half = rows_per // 2
        c1 = pltpu.async_copy(
            xt_hbm.at[:, pl.ds(row0, half)], xts.at[:, pl.ds(0, half)], s1)
        c2 = pltpu.async_copy(
            xt_hbm.at[:, pl.ds(row0 + half, half)],
            xts.at[:, pl.ds(half, half)], s2)
        pltpu.sync_copy(w_hbm, tab.at[pl.ds(0, 3)])
        t = jnp.exp(tab[...])
        addr_col = lax.iota(jnp.int32, _LANES) * n_cols

        hc = n_chunks // 2

        def compute(i, k_ofs):
            j = i // hc
            k = (i & (hc - 1)) + k_ofs
            idx = xts[j, pl.ds(k * _LANES, _LANES)]
            vals = t.at[idx].get(mode="promise_in_bounds")
            addr = addr_col + (k * (_LANES * n_cols) + j)
            plsc.store_scatter(ov, [addr], vals)

        n_work = n_cols * hc
        c1.wait()

        @plsc.parallel_loop(0, n_work, 1)
        def body1(i):
            compute(i, 0)

        c3 = pltpu.async_copy(
            ov.at[pl.ds(0, per // 2)],
            out_hbm.at[pl.ds(row0 * n_cols, per // 2)], s3)
        c2.wait()

        @plsc.parallel_loop(0, n_work, 1)
        def body2(i):
            compute(i, hc)

        c3.wait()
        pltpu.sync_copy(
            ov.at[pl.ds(per // 2, per // 2)],
            out_hbm.at[pl.ds(row0 * n_cols + per // 2, per // 2)])

    return run


def kernel(x, weight):
    n_rows, n_cols = x.shape
    return _build(n_rows, n_cols, 32)(
        x.T.astype(jnp.int32), weight.astype(jnp.float32))


# speedup vs baseline: 1.0790x; 1.0790x over previous
import functools

import jax
import jax.numpy as jnp
from jax import lax
from jax.experimental import pallas as pl
from jax.experimental.pallas import tpu as pltpu
from jax.experimental.pallas import tpu_sc as plsc

_LANES = 16


def _build(n_rows: int, n_cols: int, n_workers: int):
    rows_per = n_rows // n_workers
    per = rows_per * n_cols
    n_chunks = rows_per // _LANES
    assert rows_per * n_workers == n_rows and n_chunks * _LANES == rows_per
    mesh = plsc.VectorSubcoreMesh(core_axis_name="c", subcore_axis_name="s")

    @functools.partial(
        pl.kernel,
        out_type=jax.ShapeDtypeStruct((n_rows * n_cols,), jnp.float32),
        mesh=mesh,
        scratch_types=[
            pltpu.VMEM((_LANES,), jnp.float32),
            pltpu.VMEM((n_cols, rows_per), jnp.int32),
            pltpu.VMEM((per,), jnp.float32),
            pltpu.SemaphoreType.DMA,
        ],
        compiler_params=pltpu.CompilerParams(
            needs_layout_passes=False, skip_device_barrier=True),
    )
    def run(xt_hbm, w_hbm, out_hbm, tab, xts, ov, s1):
        wid = lax.axis_index("s") * 2 + lax.axis_index("c")
        row0 = wid * rows_per
        c1 = pltpu.async_copy(
            xt_hbm.at[:, pl.ds(row0, rows_per)], xts, s1)
        pltpu.sync_copy(w_hbm, tab.at[pl.ds(0, 3)])
        t = jnp.exp(tab[...])
        addr_col = lax.iota(jnp.int32, _LANES) * n_cols
        c1.wait()

        @plsc.parallel_loop(0, n_chunks, 1)
        def body(k):
            base = k * (_LANES * n_cols)
            for j in range(n_cols):
                idx = xts[j, pl.ds(k * _LANES, _LANES)]
                vals = t.at[idx].get(mode="promise_in_bounds")
                plsc.store_scatter(ov, [addr_col + (base + j)], vals)

        pltpu.sync_copy(ov, out_hbm.at[pl.ds(row0 * n_cols, per)])

    return run


def kernel(x, weight):
    n_rows, n_cols = x.shape
    return _build(n_rows, n_cols, 32)(
        x.T.astype(jnp.int32), weight.astype(jnp.float32))


# speedup vs baseline: 1.0863x; 1.0068x over previous
import functools

import jax
import jax.numpy as jnp
from jax import lax
from jax.experimental import pallas as pl
from jax.experimental.pallas import tpu as pltpu
from jax.experimental.pallas import tpu_sc as plsc

_LANES = 16


def _build(n_rows: int, n_cols: int, n_workers: int):
    rows_per = n_rows // n_workers
    per = rows_per * n_cols
    n_chunks = rows_per // _LANES
    assert rows_per * n_workers == n_rows and n_chunks * _LANES == rows_per
    mesh = plsc.VectorSubcoreMesh(core_axis_name="c", subcore_axis_name="s")

    @functools.partial(
        pl.kernel,
        out_type=jax.ShapeDtypeStruct((n_rows * n_cols,), jnp.float32),
        mesh=mesh,
        scratch_types=[
            pltpu.VMEM((_LANES,), jnp.float32),
            pltpu.VMEM((n_cols, rows_per), jnp.int32),
            pltpu.VMEM((per,), jnp.float32),
            pltpu.SemaphoreType.DMA,
        ],
        compiler_params=pltpu.CompilerParams(
            needs_layout_passes=False, skip_device_barrier=True),
    )
    def run(xt_hbm, w_hbm, out_hbm, tab, xts, ov, s1):
        wid = lax.axis_index("s") * 2 + lax.axis_index("c")
        row0 = wid * rows_per
        c1 = pltpu.async_copy(
            xt_hbm.at[:, pl.ds(row0, rows_per)], xts, s1)
        pltpu.sync_copy(w_hbm, tab.at[pl.ds(0, 3)])
        t = jnp.exp(tab[...])
        addr_col = lax.iota(jnp.int32, _LANES) * n_cols
        c1.wait()

        @plsc.parallel_loop(0, n_chunks, 1)
        def body(k):
            ock = ov.at[pl.ds(k * (_LANES * n_cols), _LANES * n_cols)]
            for j in range(n_cols):
                idx = xts[j, pl.ds(k * _LANES, _LANES)]
                vals = t.at[idx].get(mode="promise_in_bounds")
                plsc.store_scatter(ock, [addr_col + j], vals)

        pltpu.sync_copy(ov, out_hbm.at[pl.ds(row0 * n_cols, per)])

    return run


def kernel(x, weight):
    n_rows, n_cols = x.shape
    return _build(n_rows, n_cols, 32)(
        x.T.astype(jnp.int32), weight.astype(jnp.float32))


# speedup vs baseline: 1.0870x; 1.0006x over previous
import functools

import jax
import jax.numpy as jnp
from jax import lax
from jax.experimental import pallas as pl
from jax.experimental.pallas import tpu as pltpu
from jax.experimental.pallas import tpu_sc as plsc

_LANES = 16


def _build(n_rows: int, n_cols: int, n_workers: int):
    rows_per = n_rows // n_workers
    per = rows_per * n_cols
    n_chunks = rows_per // _LANES
    assert rows_per * n_workers == n_rows and n_chunks * _LANES == rows_per
    mesh = plsc.VectorSubcoreMesh(core_axis_name="c", subcore_axis_name="s")

    @functools.partial(
        pl.kernel,
        out_type=jax.ShapeDtypeStruct((n_rows * n_cols,), jnp.float32),
        mesh=mesh,
        scratch_types=[
            pltpu.VMEM((_LANES,), jnp.float32),
            pltpu.VMEM((n_cols, rows_per), jnp.int32),
            pltpu.VMEM((per,), jnp.float32),
            pltpu.SemaphoreType.DMA,
        ],
        compiler_params=pltpu.CompilerParams(needs_layout_passes=False),
    )
    def run(xt_hbm, w_hbm, out_hbm, tab, xts, ov, s1):
        wid = lax.axis_index("s") * 2 + lax.axis_index("c")
        row0 = wid * rows_per
        c1 = pltpu.async_copy(
            xt_hbm.at[:, pl.ds(row0, rows_per)], xts, s1)
        pltpu.sync_copy(w_hbm, tab.at[pl.ds(0, 3)])
        t = jnp.exp(tab[...])
        addr_col = lax.iota(jnp.int32, _LANES) * n_cols
        c1.wait()

        @plsc.parallel_loop(0, n_chunks, 1)
        def body(k):
            ock = ov.at[pl.ds(k * (_LANES * n_cols), _LANES * n_cols)]
            for j in range(n_cols):
                idx = xts[j, pl.ds(k * _LANES, _LANES)]
                vals = t.at[idx].get(mode="promise_in_bounds")
                plsc.store_scatter(ock, [addr_col + j], vals)

        pltpu.sync_copy(ov, out_hbm.at[pl.ds(row0 * n_cols, per)])

    return run


def kernel(x, weight):
    n_rows, n_cols = x.shape
    return _build(n_rows, n_cols, 32)(
        x.T.astype(jnp.int32), weight.astype(jnp.float32))
